# baseline (speedup 1.0000x reference)
import jax
import jax.numpy as jnp
from jax import lax
from jax.experimental import pallas as pl
from jax.experimental.pallas import tpu as pltpu
from jax.experimental.pallas import tpu_sc as plsc

_NUM_CORES = 2
_NUM_SUBCORES = 16
_GCHUNK = 32
_GROUP = 5
_NSLOT = 5
_GROWS = _GCHUNK * _GROUP


def _sc_gather(W, idx_flat, n, D):
    mesh = plsc.VectorSubcoreMesh(core_axis_name="c", subcore_axis_name="s")
    nw = _NUM_CORES * _NUM_SUBCORES
    per_w = n // nw
    groups = per_w // _GROWS
    iters = groups // _NSLOT

    @pl.kernel(
        out_type=jax.ShapeDtypeStruct((n, D), W.dtype),
        mesh=mesh,
        scratch_types=[
            pltpu.VMEM((per_w,), jnp.int32),
            pltpu.VMEM((_NSLOT, _GROWS, D), W.dtype),
            pltpu.SemaphoreType.DMA((_NSLOT, _GROUP)),
            pltpu.SemaphoreType.DMA((_NSLOT,)),
            pltpu.SemaphoreType.DMA,
        ],
    )
    def gather_kernel(w_hbm, i_hbm, o_hbm, idx_v, buf, gsem, wsem, isem):
        wid = lax.axis_index("s") * _NUM_CORES + lax.axis_index("c")
        base = wid * per_w
        pltpu.async_copy(i_hbm.at[pl.ds(base, per_w)], idx_v, isem).wait()

        def gather_group(g, s):
            for j in range(_GROUP):
                src = w_hbm.at[idx_v.at[pl.ds(g * _GROWS + j * _GCHUNK, _GCHUNK)]]
                dst = buf.at[s, pl.ds(j * _GCHUNK, _GCHUNK)]
                pltpu.make_async_copy(src, dst, gsem.at[s, j]).start()

        def wait_group(g, s):
            for j in range(_GROUP):
                src = w_hbm.at[idx_v.at[pl.ds(g * _GROWS + j * _GCHUNK, _GCHUNK)]]
                dst = buf.at[s, pl.ds(j * _GCHUNK, _GCHUNK)]
                pltpu.make_async_copy(src, dst, gsem.at[s, j]).wait()

        def write_group(g, s):
            dst = o_hbm.at[pl.ds(base + g * _GROWS, _GROWS)]
            pltpu.make_async_copy(buf.at[s], dst, wsem.at[s]).start()

        def wait_write(g, s):
            dst = o_hbm.at[pl.ds(base + g * _GROWS, _GROWS)]
            pltpu.make_async_copy(buf.at[s], dst, wsem.at[s]).wait()

        for s in range(_NSLOT):
            gather_group(jnp.int32(s), s)

        @pl.loop(0, iters - 1)
        def _(k):
            g0 = k * _NSLOT
            for s in range(_NSLOT):
                wait_group(g0 + s, s)
                write_group(g0 + s, s)
            for s in range(_NSLOT):
                wait_write(g0 + s, s)
                gather_group(g0 + _NSLOT + s, s)

        gl = jnp.int32((iters - 1) * _NSLOT)
        for s in range(_NSLOT):
            wait_group(gl + s, s)
            write_group(gl + s, s)
        for s in range(_NSLOT):
            wait_write(gl + s, s)

    return gather_kernel(W, idx_flat)


def kernel(x, W):
    B, L = x.shape
    V, D = W.shape
    n = B * L
    idx_flat = x.reshape(n).astype(jnp.int32)
    out = _sc_gather(W, idx_flat, n, D)
    return out.reshape(B, L, D)


# speedup vs baseline: 1.0254x; 1.0254x over previous
import jax
import jax.numpy as jnp
from jax import lax
from jax.experimental import pallas as pl
from jax.experimental.pallas import tpu as pltpu
from jax.experimental.pallas import tpu_sc as plsc

_NUM_CORES = 2
_NUM_SUBCORES = 16
_GCHUNK = 64
_GROUP = 2
_NSLOT = 5
_GROWS = _GCHUNK * _GROUP


def _sc_gather(W, idx_flat, n, D):
    mesh = plsc.VectorSubcoreMesh(core_axis_name="c", subcore_axis_name="s")
    nw = _NUM_CORES * _NUM_SUBCORES
    per_w = n // nw
    groups = per_w // _GROWS
    iters = groups // _NSLOT

    @pl.kernel(
        out_type=jax.ShapeDtypeStruct((n, D), W.dtype),
        mesh=mesh,
        scratch_types=[
            pltpu.VMEM((per_w,), jnp.int32),
            pltpu.VMEM((_NSLOT, _GROWS, D), W.dtype),
            pltpu.SemaphoreType.DMA((_NSLOT, _GROUP)),
            pltpu.SemaphoreType.DMA((_NSLOT,)),
            pltpu.SemaphoreType.DMA,
        ],
    )
    def gather_kernel(w_hbm, i_hbm, o_hbm, idx_v, buf, gsem, wsem, isem):
        wid = lax.axis_index("s") * _NUM_CORES + lax.axis_index("c")
        base = wid * per_w
        pltpu.async_copy(i_hbm.at[pl.ds(base, per_w)], idx_v, isem).wait()

        def gather_group(g, s):
            for j in range(_GROUP):
                src = w_hbm.at[idx_v.at[pl.ds(g * _GROWS + j * _GCHUNK, _GCHUNK)]]
                dst = buf.at[s, pl.ds(j * _GCHUNK, _GCHUNK)]
                pltpu.make_async_copy(src, dst, gsem.at[s, j]).start()

        def wait_group(g, s):
            for j in range(_GROUP):
                src = w_hbm.at[idx_v.at[pl.ds(g * _GROWS + j * _GCHUNK, _GCHUNK)]]
                dst = buf.at[s, pl.ds(j * _GCHUNK, _GCHUNK)]
                pltpu.make_async_copy(src, dst, gsem.at[s, j]).wait()

        def write_group(g, s):
            dst = o_hbm.at[pl.ds(base + g * _GROWS, _GROWS)]
            pltpu.make_async_copy(buf.at[s], dst, wsem.at[s]).start()

        def wait_write(g, s):
            dst = o_hbm.at[pl.ds(base + g * _GROWS, _GROWS)]
            pltpu.make_async_copy(buf.at[s], dst, wsem.at[s]).wait()

        for s in range(_NSLOT):
            gather_group(jnp.int32(s), s)

        @pl.loop(0, iters - 1)
        def _(k):
            g0 = k * _NSLOT
            for s in range(_NSLOT):
                wait_group(g0 + s, s)
                write_group(g0 + s, s)
            for s in range(_NSLOT):
                wait_write(g0 + s, s)
                gather_group(g0 + _NSLOT + s, s)

        gl = jnp.int32((iters - 1) * _NSLOT)
        for s in range(_NSLOT):
            wait_group(gl + s, s)
            write_group(gl + s, s)
        for s in range(_NSLOT):
            wait_write(gl + s, s)

    return gather_kernel(W, idx_flat)


def kernel(x, W):
    B, L = x.shape
    V, D = W.shape
    n = B * L
    idx_flat = x.reshape(n).astype(jnp.int32)
    out = _sc_gather(W, idx_flat, n, D)
    return out.reshape(B, L, D)
